# Initial kernel scaffold; baseline (speedup 1.0000x reference)
import jax
import jax.numpy as jnp
from jax.experimental import pallas as pl


def kernel(x, edge_attr, params, edge_index, batch):
    raise NotImplementedError("write your pallas kernel here")


# speedup vs baseline: 10.1345x; 10.1345x over previous
import functools

import jax
import jax.numpy as jnp
from jax import lax
from jax.experimental import pallas as pl
from jax.experimental.pallas import tpu as pltpu
from jax.experimental.pallas import tpu_sc as plsc

N = 10000
NP = 10240
E = 160000
D = 128
HEADS = 6
HC = 64
NG = 64
NCLS = 10
KPROP = 5
ALPHA = 0.1
HP = 16
XSW = HEADS * HC

NC = 2
NS = 16
NW = NC * NS
EPAD = 163840
EW_PER_TILE = EPAD // NW
ROWS_PER_TILE = NP // NS
NEG = -1e9

_MESH = plsc.VectorSubcoreMesh(core_axis_name="c", subcore_axis_name="s")
_f32 = jnp.float32
_i32 = jnp.int32


def _lane_iota():
  return lax.broadcasted_iota(_i32, (16,), 0)


def _splat(v):
  return jnp.full((16,), v, _i32)


def _core_ids():
  c = lax.axis_index("c")
  s = lax.axis_index("s")
  return c, s, (s * NC + c) * EW_PER_TILE


def _zero_acc(zeros_hbm, acc, s):
  pltpu.sync_copy(zeros_hbm, acc.at[pl.ds(s * ROWS_PER_TILE, ROWS_PER_TILE)])
  plsc.subcore_barrier()


def _acc_out(acc, out, c, s):
  plsc.subcore_barrier()
  r0 = s * ROWS_PER_TILE
  pltpu.sync_copy(acc.at[pl.ds(r0, ROWS_PER_TILE)],
                  out.at[c, pl.ds(r0, ROWS_PER_TILE)])


@functools.partial(
    pl.kernel, mesh=_MESH,
    compiler_params=pltpu.CompilerParams(needs_layout_passes=False, use_tc_tiling_on_sc=False),
    out_type=jax.ShapeDtypeStruct((NC, NP, HP), _f32),
    scratch_types=[
        pltpu.VMEM((128,), _i32),
        pltpu.VMEM((128,), _f32),
        pltpu.VMEM((128, HP), _f32),
        pltpu.VMEM_SHARED((NP, HP), _f32),
    ],
)
def _sc_deg(col_hbm, ew_hbm, zeros_hbm, out_hbm, cidx, ewv, sbuf, acc):
  c, s, base = _core_ids()
  _zero_acc(zeros_hbm, acc, s)

  @pl.loop(0, EW_PER_TILE // 128)
  def _(j):
    off = base + j * 128
    pltpu.sync_copy(col_hbm.at[pl.ds(off, 128)], cidx)
    pltpu.sync_copy(ew_hbm.at[pl.ds(off, 128)], ewv)

    @pl.loop(0, 128)
    def _(e):
      w = plsc.load_gather(ewv, [_splat(e)])
      unit = jnp.where(_lane_iota() == 0, 1.0, 0.0).astype(_f32)
      sbuf[e, pl.ds(0, HP)] = w * unit

    pltpu.sync_copy(sbuf, acc.at[cidx], add=True)

  _acc_out(acc, out_hbm, c, s)


@functools.partial(
    pl.kernel, mesh=_MESH,
    compiler_params=pltpu.CompilerParams(needs_layout_passes=False, use_tc_tiling_on_sc=False),
    out_type=jax.ShapeDtypeStruct((EPAD,), _f32),
    scratch_types=[
        pltpu.VMEM((128,), _i32),
        pltpu.VMEM((128,), _i32),
        pltpu.VMEM((128,), _f32),
        pltpu.VMEM((128, HP), _f32),
        pltpu.VMEM((128, HP), _f32),
        pltpu.VMEM((128,), _f32),
        pltpu.SemaphoreType.DMA,
        pltpu.SemaphoreType.DMA,
    ],
)
def _sc_norm(row_hbm, col_hbm, ew_hbm, dinv_hbm, out_hbm,
             ridx, cidx, ewv, dvr, dvc, nbuf, sem1, sem2):
  _, _, base = _core_ids()

  @pl.loop(0, EW_PER_TILE // 128)
  def _(j):
    off = base + j * 128
    pltpu.sync_copy(row_hbm.at[pl.ds(off, 128)], ridx)
    pltpu.sync_copy(col_hbm.at[pl.ds(off, 128)], cidx)
    pltpu.sync_copy(ew_hbm.at[pl.ds(off, 128)], ewv)
    cp1 = pltpu.async_copy(dinv_hbm.at[ridx], dvr, sem1)
    cp2 = pltpu.async_copy(dinv_hbm.at[cidx], dvc, sem2)
    cp1.wait()
    cp2.wait()

    @pl.loop(0, 8)
    def _(k):
      lanes = _lane_iota()
      rows = lanes + k * 16
      a = plsc.load_gather(dvr, [rows, lanes])
      b = plsc.load_gather(dvc, [rows, lanes])
      nbuf[pl.ds(k * 16, 16)] = a * ewv[pl.ds(k * 16, 16)] * b

    pltpu.sync_copy(nbuf, out_hbm.at[pl.ds(off, 128)])


@functools.partial(
    pl.kernel, mesh=_MESH,
    compiler_params=pltpu.CompilerParams(needs_layout_passes=False, use_tc_tiling_on_sc=False),
    out_type=jax.ShapeDtypeStruct((NC, NP, D), _f32),
    scratch_types=[
        pltpu.VMEM((128,), _i32),
        pltpu.VMEM((128,), _i32),
        pltpu.VMEM((128,), _f32),
        pltpu.VMEM((128, D), _f32),
        pltpu.VMEM((128, D), _f32),
        pltpu.VMEM_SHARED((NP, D), _f32),
        pltpu.SemaphoreType.DMA,
    ],
)
def _sc_appnp(row_hbm, col_hbm, nrm_hbm, x_hbm, zeros_hbm, out_hbm,
              ridx, cidx, nrm, rows, sbuf, acc, sem):
  c, s, base = _core_ids()
  _zero_acc(zeros_hbm, acc, s)

  @pl.loop(0, EW_PER_TILE // 128)
  def _(j):
    off = base + j * 128
    pltpu.sync_copy(row_hbm.at[pl.ds(off, 128)], ridx)
    pltpu.sync_copy(col_hbm.at[pl.ds(off, 128)], cidx)
    pltpu.sync_copy(nrm_hbm.at[pl.ds(off, 128)], nrm)
    pltpu.async_copy(x_hbm.at[ridx], rows, sem).wait()

    @pl.loop(0, 128)
    def _(e):
      w = plsc.load_gather(nrm, [_splat(e)])
      for k in range(D // 16):
        sbuf[e, pl.ds(k * 16, 16)] = w * rows[e, pl.ds(k * 16, 16)]

    pltpu.sync_copy(sbuf, acc.at[cidx], add=True)

  _acc_out(acc, out_hbm, c, s)


@functools.partial(
    pl.kernel, mesh=_MESH,
    compiler_params=pltpu.CompilerParams(needs_layout_passes=False, use_tc_tiling_on_sc=False),
    out_type=(jax.ShapeDtypeStruct((EPAD, HP), _f32),
              jax.ShapeDtypeStruct((NC, NP, HP), _f32)),
    scratch_types=[
        pltpu.VMEM((128,), _i32),
        pltpu.VMEM((128,), _i32),
        pltpu.VMEM((128, HP), _f32),
        pltpu.VMEM((128, HP), _f32),
        pltpu.VMEM((128, HP), _f32),
        pltpu.VMEM((128, HP), _f32),
        pltpu.VMEM_SHARED((NP, HP), _f32),
        pltpu.SemaphoreType.DMA,
        pltpu.SemaphoreType.DMA,
    ],
)
def _sc_gat_a(row_hbm, col_hbm, ae_hbm, asrc_hbm, adst_hbm, zeros_hbm,
              s_hbm, den_hbm,
              ridx, cidx, aev, av, bv, sbuf, acc, sem1, sem2):
  c, s, base = _core_ids()
  _zero_acc(zeros_hbm, acc, s)

  @pl.loop(0, EW_PER_TILE // 128)
  def _(j):
    off = base + j * 128
    pltpu.sync_copy(row_hbm.at[pl.ds(off, 128)], ridx)
    pltpu.sync_copy(col_hbm.at[pl.ds(off, 128)], cidx)
    pltpu.sync_copy(ae_hbm.at[pl.ds(off, 128)], aev)
    cp1 = pltpu.async_copy(asrc_hbm.at[ridx], av, sem1)
    cp2 = pltpu.async_copy(adst_hbm.at[cidx], bv, sem2)
    cp1.wait()
    cp2.wait()

    @pl.loop(0, 128)
    def _(e):
      a = av[e, pl.ds(0, 16)] + bv[e, pl.ds(0, 16)] + aev[e, pl.ds(0, 16)]
      a = jnp.maximum(a, 0.2 * a)
      sbuf[e, pl.ds(0, 16)] = jnp.exp(a)

    pltpu.sync_copy(sbuf, s_hbm.at[pl.ds(off, 128)])
    pltpu.sync_copy(sbuf, acc.at[cidx], add=True)

  _acc_out(acc, den_hbm, c, s)


@functools.partial(
    pl.kernel, mesh=_MESH,
    compiler_params=pltpu.CompilerParams(needs_layout_passes=False, use_tc_tiling_on_sc=False),
    out_type=jax.ShapeDtypeStruct((NC, NP, HC), _f32),
    scratch_types=[
        pltpu.VMEM((64,), _i32),
        pltpu.VMEM((64,), _i32),
        pltpu.VMEM((64, HP), _f32),
        pltpu.VMEM((64, HP), _f32),
        pltpu.VMEM((64, XSW), _f32),
        pltpu.VMEM((16,), _f32),
        pltpu.VMEM((64, HC), _f32),
        pltpu.VMEM_SHARED((NP, HC), _f32),
        pltpu.SemaphoreType.DMA,
        pltpu.SemaphoreType.DMA,
    ],
)
def _sc_gat_b(row_hbm, col_hbm, s_hbm, dfin_hbm, xs_hbm, zeros_hbm, out_hbm,
              ridx, cidx, sv, dv, xrows, asc, mbuf, acc, sem1, sem2):
  c, s, base = _core_ids()
  _zero_acc(zeros_hbm, acc, s)

  @pl.loop(0, EW_PER_TILE // 64)
  def _(j):
    off = base + j * 64
    pltpu.sync_copy(row_hbm.at[pl.ds(off, 64)], ridx)
    pltpu.sync_copy(col_hbm.at[pl.ds(off, 64)], cidx)
    pltpu.sync_copy(s_hbm.at[pl.ds(off, 64)], sv)
    cp1 = pltpu.async_copy(dfin_hbm.at[cidx], dv, sem1)
    cp2 = pltpu.async_copy(xs_hbm.at[ridx], xrows, sem2)
    cp1.wait()
    cp2.wait()

    @pl.loop(0, 64)
    def _(e):
      attn = sv[e, pl.ds(0, 16)] / dv[e, pl.ds(0, 16)]
      asc[pl.ds(0, 16)] = attn
      m = [jnp.zeros((16,), _f32) for _ in range(HC // 16)]
      for h in range(HEADS):
        w = plsc.load_gather(asc, [_splat(h)])
        for k in range(HC // 16):
          m[k] = m[k] + w * xrows[e, pl.ds(h * HC + k * 16, 16)]
      for k in range(HC // 16):
        mbuf[e, pl.ds(k * 16, 16)] = m[k]

    pltpu.sync_copy(mbuf, acc.at[cidx], add=True)

  _acc_out(acc, out_hbm, c, s)


def _tc_edge_body(wdict_n, ea_ref, *refs):
  w = {k: r[...] for k, r in zip(wdict_n, refs[:len(wdict_n)])}
  ew_ref, ae1_ref, ae2_ref, easum_ref = refs[len(wdict_n):]
  i = pl.program_id(0)
  ea = ea_ref[...]
  bs = ea.shape[0]
  rid = i * bs + lax.broadcasted_iota(_i32, (bs, 1), 0)
  valid = rid < E

  def mlp2(z, w1, b1, w2, b2, act):
    z1 = act(z[:, 0:1] * w1[0:1, :] + z[:, 1:2] * w1[1:2, :] + b1)
    return jnp.dot(z1, w2, preferred_element_type=_f32) + b2

  ew = jax.nn.sigmoid(
      mlp2(ea, w['ew_w1'], w['ew_b1'], w['ew_w2'], w['ew_b2'],
           jax.nn.sigmoid))
  ew_ref[...] = jnp.where(valid, ew, 0.0)

  for (w1, b1, w2, b2, wef, oref) in (
      (w['ea1_w1'], w['ea1_b1'], w['ea1_w2'], w['ea1_b2'], w['we1f'],
       ae1_ref),
      (w['ea2_w1'], w['ea2_b1'], w['ea2_w2'], w['ea2_b2'], w['we2f'],
       ae2_ref)):
    eax = mlp2(ea, w1, b1, w2, b2, jax.nn.relu)
    ae = jnp.dot(eax, wef, preferred_element_type=_f32)
    ae = jnp.concatenate([ae, jnp.full((bs, HP - HEADS), NEG, _f32)], axis=1)
    oref[...] = jnp.where(valid, ae, NEG)

  @pl.when(i == 0)
  def _():
    easum_ref[...] = jnp.zeros_like(easum_ref)
  easum_ref[...] += jnp.sum(jnp.where(valid, ea, 0.0), axis=0, keepdims=True)


def _tc_edge(ea_pad, wdict):
  names = list(wdict)
  bs = 4096
  grid = EPAD // bs
  wspecs = [pl.BlockSpec(wdict[k].shape,
                         functools.partial(lambda nd, i: (0,) * nd,
                                           wdict[k].ndim))
            for k in names]
  return pl.pallas_call(
      functools.partial(_tc_edge_body, names),
      grid=(grid,),
      in_specs=[pl.BlockSpec((bs, 2), lambda i: (i, 0))] + wspecs,
      out_specs=[
          pl.BlockSpec((bs, 1), lambda i: (i, 0)),
          pl.BlockSpec((bs, HP), lambda i: (i, 0)),
          pl.BlockSpec((bs, HP), lambda i: (i, 0)),
          pl.BlockSpec((1, 2), lambda i: (0, 0)),
      ],
      out_shape=[
          jax.ShapeDtypeStruct((EPAD, 1), _f32),
          jax.ShapeDtypeStruct((EPAD, HP), _f32),
          jax.ShapeDtypeStruct((EPAD, HP), _f32),
          jax.ShapeDtypeStruct((1, 2), _f32),
      ],
  )(ea_pad, *[wdict[k] for k in names])


def _tc_deg_body(d0_ref, d1_ref, dinv_ref, selfw_ref):
  deg = d0_ref[:, 0:1] + d1_ref[:, 0:1] + 1.0
  dinv_ref[...] = jnp.broadcast_to(lax.rsqrt(deg), dinv_ref.shape)
  selfw_ref[...] = jnp.broadcast_to(1.0 / deg, selfw_ref.shape)


def _tc_deg(d0, d1):
  bs = 1024
  return pl.pallas_call(
      _tc_deg_body,
      grid=(NP // bs,),
      in_specs=[pl.BlockSpec((bs, HP), lambda i: (i, 0))] * 2,
      out_specs=[pl.BlockSpec((bs, HP), lambda i: (i, 0))] * 2,
      out_shape=[jax.ShapeDtypeStruct((NP, HP), _f32)] * 2,
  )(d0, d1)


def _tc_appnp_combine_body(final_relu, p0_ref, p1_ref, xin_ref, h0_ref,
                           selfw_ref, out_ref):
  agg = p0_ref[...] + p1_ref[...] + xin_ref[...] * selfw_ref[:, 0:1]
  out = (1.0 - ALPHA) * agg + ALPHA * h0_ref[...]
  if final_relu:
    out = jnp.maximum(out, 0.0)
  out_ref[...] = out


def _tc_appnp_combine(p0, p1, xin, h0, selfw, final_relu):
  bs = 1024
  return pl.pallas_call(
      functools.partial(_tc_appnp_combine_body, final_relu),
      grid=(NP // bs,),
      in_specs=[pl.BlockSpec((bs, D), lambda i: (i, 0))] * 4 +
               [pl.BlockSpec((bs, HP), lambda i: (i, 0))],
      out_specs=pl.BlockSpec((bs, D), lambda i: (i, 0)),
      out_shape=jax.ShapeDtypeStruct((NP, D), _f32),
  )(p0, p1, xin, h0, selfw)


def _tc_gat_node_body(h_ref, ws_ref, wd_ref, as_ref, ad_ref,
                      xs_ref, asrc_ref, adst_ref):
  h = h_ref[...]
  xs = jnp.dot(h, ws_ref[...], preferred_element_type=_f32)
  xd = jnp.dot(h, wd_ref[...], preferred_element_type=_f32)
  xs_ref[...] = xs
  bs = h.shape[0]

  def headsum(v, a):
    cols = [jnp.sum(v[:, hh * HC:(hh + 1) * HC] * a[:, hh * HC:(hh + 1) * HC],
                    axis=1, keepdims=True) for hh in range(HEADS)]
    cols.append(jnp.zeros((bs, HP - HEADS), _f32))
    return jnp.concatenate(cols, axis=1)

  asrc_ref[...] = headsum(xs, as_ref[...])
  adst_ref[...] = headsum(xd, ad_ref[...])


def _tc_gat_node(h, ws, wd, a_s, a_d):
  bs = 1024
  din = h.shape[1]
  return pl.pallas_call(
      _tc_gat_node_body,
      grid=(NP // bs,),
      in_specs=[
          pl.BlockSpec((bs, din), lambda i: (i, 0)),
          pl.BlockSpec((din, XSW), lambda i: (0, 0)),
          pl.BlockSpec((din, XSW), lambda i: (0, 0)),
          pl.BlockSpec((1, XSW), lambda i: (0, 0)),
          pl.BlockSpec((1, XSW), lambda i: (0, 0)),
      ],
      out_specs=[
          pl.BlockSpec((bs, XSW), lambda i: (i, 0)),
          pl.BlockSpec((bs, HP), lambda i: (i, 0)),
          pl.BlockSpec((bs, HP), lambda i: (i, 0)),
      ],
      out_shape=[
          jax.ShapeDtypeStruct((NP, XSW), _f32),
          jax.ShapeDtypeStruct((NP, HP), _f32),
          jax.ShapeDtypeStruct((NP, HP), _f32),
      ],
  )(h, ws, wd, a_s, a_d)


def _tc_denom_body(d0_ref, d1_ref, asrc_ref, adst_ref, easum_ref,
                   w1_ref, b1_ref, w2_ref, b2_ref, wef_ref,
                   dfin_ref, sself_ref):
  la = easum_ref[...] / float(E)
  z1 = jax.nn.relu(la[:, 0:1] * w1_ref[0:1, :] + la[:, 1:2] * w1_ref[1:2, :]
                   + b1_ref[...])
  z2 = jnp.dot(z1, w2_ref[...], preferred_element_type=_f32) + b2_ref[...]
  aes = jnp.dot(z2, wef_ref[...], preferred_element_type=_f32)
  aes = jnp.concatenate(
      [aes, jnp.full((1, HP - HEADS), NEG, _f32)], axis=1)
  a = asrc_ref[...] + adst_ref[...] + aes
  sself = jnp.exp(jnp.maximum(a, 0.2 * a))
  sself_ref[...] = sself
  dfin_ref[...] = d0_ref[...] + d1_ref[...] + sself + 1e-16


def _tc_denom(d0, d1, asrc, adst, easum, w1, b1, w2, b2, wef):
  bs = 1024
  return pl.pallas_call(
      _tc_denom_body,
      grid=(NP // bs,),
      in_specs=[pl.BlockSpec((bs, HP), lambda i: (i, 0))] * 4 + [
          pl.BlockSpec((1, 2), lambda i: (0, 0)),
          pl.BlockSpec((2, 8), lambda i: (0, 0)),
          pl.BlockSpec((1, 8), lambda i: (0, 0)),
          pl.BlockSpec((8, 4), lambda i: (0, 0)),
          pl.BlockSpec((1, 4), lambda i: (0, 0)),
          pl.BlockSpec((4, HEADS), lambda i: (0, 0)),
      ],
      out_specs=[pl.BlockSpec((bs, HP), lambda i: (i, 0))] * 2,
      out_shape=[jax.ShapeDtypeStruct((NP, HP), _f32)] * 2,
  )(d0, d1, asrc, adst, easum, w1, b1, w2, b2, wef)


def _tc_gat_final_body(p0_ref, p1_ref, sself_ref, dfin_ref, xs_ref, b_ref,
                       out_ref):
  acc = p0_ref[...] + p1_ref[...]
  sself = sself_ref[...]
  dfin = dfin_ref[...]
  xs = xs_ref[...]
  for h in range(HEADS):
    w = sself[:, h:h + 1] / dfin[:, h:h + 1]
    acc = acc + w * xs[:, h * HC:(h + 1) * HC]
  out_ref[...] = jnp.maximum(acc * (1.0 / HEADS) + b_ref[...], 0.0)


def _tc_gat_final(p0, p1, sself, dfin, xs, bias):
  bs = 1024
  return pl.pallas_call(
      _tc_gat_final_body,
      grid=(NP // bs,),
      in_specs=[
          pl.BlockSpec((bs, HC), lambda i: (i, 0)),
          pl.BlockSpec((bs, HC), lambda i: (i, 0)),
          pl.BlockSpec((bs, HP), lambda i: (i, 0)),
          pl.BlockSpec((bs, HP), lambda i: (i, 0)),
          pl.BlockSpec((bs, XSW), lambda i: (i, 0)),
          pl.BlockSpec((1, HC), lambda i: (0, 0)),
      ],
      out_specs=pl.BlockSpec((bs, HC), lambda i: (i, 0)),
      out_shape=jax.ShapeDtypeStruct((NP, HC), _f32),
  )(p0, p1, sself, dfin, xs, bias)


def _tc_pool_body(h_ref, batch_ref, lw_ref, lb_ref, out_ref):
  h = h_ref[...]
  b = batch_ref[...]
  ids = lax.broadcasted_iota(_i32, (1, NG), 1)
  oh = (b == ids).astype(_f32)
  sums = lax.dot_general(oh, h, (((0,), (0,)), ((), ())),
                         preferred_element_type=_f32)
  ones = jnp.ones((NP, 1), _f32)
  cnt = lax.dot_general(oh, ones, (((0,), (0,)), ((), ())),
                        preferred_element_type=_f32)
  pooled = jnp.maximum(sums / jnp.maximum(cnt, 1.0), 0.0)
  logits = jnp.dot(pooled, lw_ref[...], preferred_element_type=_f32) \
      + lb_ref[...]
  m = jnp.max(logits, axis=1, keepdims=True)
  sh = logits - m
  out_ref[...] = sh - jnp.log(jnp.sum(jnp.exp(sh), axis=1, keepdims=True))


def _tc_pool(h, batch2d, lw, lb):
  return pl.pallas_call(
      _tc_pool_body,
      grid=(1,),
      in_specs=[
          pl.BlockSpec((NP, HC), lambda i: (0, 0)),
          pl.BlockSpec((NP, 1), lambda i: (0, 0)),
          pl.BlockSpec((HC, NCLS), lambda i: (0, 0)),
          pl.BlockSpec((1, NCLS), lambda i: (0, 0)),
      ],
      out_specs=pl.BlockSpec((NG, NCLS), lambda i: (0, 0)),
      out_shape=jax.ShapeDtypeStruct((NG, NCLS), _f32),
  )(h, batch2d, lw, lb)


def kernel(x, edge_attr, params, edge_index, batch):
  p = params
  row = jnp.pad(edge_index[0], (0, EPAD - E))
  col = jnp.pad(edge_index[1], (0, EPAD - E))
  ea_pad = jnp.pad(edge_attr, ((0, EPAD - E), (0, 0)))

  def fold_we(we, a_e):
    return (we.reshape(4, HEADS, HC) * a_e[0][None]).sum(-1)

  wdict = {
      'ew_w1': p['ew_w1'], 'ew_b1': p['ew_b1'].reshape(1, 4),
      'ew_w2': p['ew_w2'], 'ew_b2': p['ew_b2'].reshape(1, 1),
      'ea1_w1': p['ea1_w1'], 'ea1_b1': p['ea1_b1'].reshape(1, 8),
      'ea1_w2': p['ea1_w2'], 'ea1_b2': p['ea1_b2'].reshape(1, 4),
      'ea2_w1': p['ea2_w1'], 'ea2_b1': p['ea2_b1'].reshape(1, 8),
      'ea2_w2': p['ea2_w2'], 'ea2_b2': p['ea2_b2'].reshape(1, 4),
      'we1f': fold_we(p['g1_we'], p['g1_ae']),
      'we2f': fold_we(p['g2_we'], p['g2_ae']),
  }
  ew2d, ae1, ae2, easum = _tc_edge(ea_pad, wdict)
  ew = ew2d.reshape(EPAD)
  xp = jnp.pad(x, ((0, NP - N), (0, 0)))
  batch_p = jnp.pad(batch.astype(_i32), (0, NP - N), constant_values=NG)

  zeros16 = jnp.zeros((ROWS_PER_TILE, HP), _f32)
  zeros128 = jnp.zeros((ROWS_PER_TILE, D), _f32)
  zeros64 = jnp.zeros((ROWS_PER_TILE, HC), _f32)

  degp = _sc_deg(col, ew, zeros16)
  dinv, selfw = _tc_deg(degp[0], degp[1])
  dinv_flat = dinv
  nrm = _sc_norm(row, col, ew, dinv_flat)

  h0 = xp
  out = xp
  for it in range(KPROP):
    part = _sc_appnp(row, col, nrm, out, zeros128)
    out = _tc_appnp_combine(part[0], part[1], out, h0, selfw,
                            final_relu=(it == KPROP - 1))
  h = out

  for (ws, wd, a_s, a_d, bias, ae, wef_n) in (
      (p['g1_ws'], p['g1_wd'], p['g1_as'], p['g1_ad'], p['g1_b'], ae1,
       ('ea1_w1', 'ea1_b1', 'ea1_w2', 'ea1_b2', 'we1f')),
      (p['g2_ws'], p['g2_wd'], p['g2_as'], p['g2_ad'], p['g2_b'], ae2,
       ('ea2_w1', 'ea2_b1', 'ea2_w2', 'ea2_b2', 'we2f'))):
    xs, asrc, adst = _tc_gat_node(h, ws, wd,
                                  a_s.reshape(1, XSW), a_d.reshape(1, XSW))
    sarr, denp = _sc_gat_a(row, col, ae, asrc, adst, zeros16)
    dfin, sself = _tc_denom(denp[0], denp[1], asrc, adst, easum,
                            wdict[wef_n[0]], wdict[wef_n[1]],
                            wdict[wef_n[2]], wdict[wef_n[3]],
                            wdict[wef_n[4]])
    part = _sc_gat_b(row, col, sarr, dfin, xs, zeros64)
    h = _tc_gat_final(part[0], part[1], sself, dfin, xs,
                      bias.reshape(1, HC))

  return _tc_pool(h, batch_p.reshape(NP, 1),
                  p['lin_w'], p['lin_b'].reshape(1, NCLS))


# speedup vs baseline: 16.7759x; 1.6553x over previous
import functools

import jax
import jax.numpy as jnp
from jax import lax
from jax.experimental import pallas as pl
from jax.experimental.pallas import tpu as pltpu
from jax.experimental.pallas import tpu_sc as plsc

N = 10000
NP = 10240
E = 160000
D = 128
HEADS = 6
HC = 64
NG = 64
NCLS = 10
KPROP = 5
ALPHA = 0.1
HP = 16
XSW = HEADS * HC

NC = 2
NS = 16
NW = NC * NS
EPAD = 163840
EW_PER_TILE = EPAD // NW
ROWS_PER_TILE = NP // NS
NEG = -1e9

_MESH = plsc.VectorSubcoreMesh(core_axis_name="c", subcore_axis_name="s")
_f32 = jnp.float32
_i32 = jnp.int32


def _lane_iota():
  return lax.broadcasted_iota(_i32, (16,), 0)


def _splat(v):
  return jnp.full((16,), v, _i32)


def _core_ids():
  c = lax.axis_index("c")
  s = lax.axis_index("s")
  return c, s, (s * NC + c) * EW_PER_TILE


def _zero_acc(zeros_hbm, acc, s):
  pltpu.sync_copy(zeros_hbm, acc.at[pl.ds(s * ROWS_PER_TILE, ROWS_PER_TILE)])
  plsc.subcore_barrier()


def _acc_out(acc, out, c, s):
  plsc.subcore_barrier()
  r0 = s * ROWS_PER_TILE
  pltpu.sync_copy(acc.at[pl.ds(r0, ROWS_PER_TILE)],
                  out.at[c, pl.ds(r0, ROWS_PER_TILE)])


@functools.partial(
    pl.kernel, mesh=_MESH,
    compiler_params=pltpu.CompilerParams(needs_layout_passes=False, use_tc_tiling_on_sc=False),
    out_type=jax.ShapeDtypeStruct((NC, NP, HP), _f32),
    scratch_types=[
        pltpu.VMEM((128,), _i32),
        pltpu.VMEM((128,), _f32),
        pltpu.VMEM((128, HP), _f32),
        pltpu.VMEM_SHARED((NP, HP), _f32),
    ],
)
def _sc_deg(col_hbm, ew_hbm, zeros_hbm, out_hbm, cidx, ewv, sbuf, acc):
  c, s, base = _core_ids()
  _zero_acc(zeros_hbm, acc, s)

  @pl.loop(0, EW_PER_TILE // 128)
  def _(j):
    off = base + j * 128
    pltpu.sync_copy(col_hbm.at[pl.ds(off, 128)], cidx)
    pltpu.sync_copy(ew_hbm.at[pl.ds(off, 128)], ewv)

    @pl.loop(0, 128)
    def _(e):
      w = plsc.load_gather(ewv, [_splat(e)])
      unit = jnp.where(_lane_iota() == 0, 1.0, 0.0).astype(_f32)
      sbuf[e, pl.ds(0, HP)] = w * unit

    pltpu.sync_copy(sbuf, acc.at[cidx], add=True)

  _acc_out(acc, out_hbm, c, s)


@functools.partial(
    pl.kernel, mesh=_MESH,
    compiler_params=pltpu.CompilerParams(needs_layout_passes=False, use_tc_tiling_on_sc=False),
    out_type=jax.ShapeDtypeStruct((EPAD,), _f32),
    scratch_types=[
        pltpu.VMEM((128,), _i32),
        pltpu.VMEM((128,), _i32),
        pltpu.VMEM((128,), _f32),
        pltpu.VMEM((128, HP), _f32),
        pltpu.VMEM((128, HP), _f32),
        pltpu.VMEM((128,), _f32),
        pltpu.SemaphoreType.DMA,
        pltpu.SemaphoreType.DMA,
    ],
)
def _sc_norm(row_hbm, col_hbm, ew_hbm, dinv_hbm, out_hbm,
             ridx, cidx, ewv, dvr, dvc, nbuf, sem1, sem2):
  _, _, base = _core_ids()

  @pl.loop(0, EW_PER_TILE // 128)
  def _(j):
    off = base + j * 128
    pltpu.sync_copy(row_hbm.at[pl.ds(off, 128)], ridx)
    pltpu.sync_copy(col_hbm.at[pl.ds(off, 128)], cidx)
    pltpu.sync_copy(ew_hbm.at[pl.ds(off, 128)], ewv)
    cp1 = pltpu.async_copy(dinv_hbm.at[ridx], dvr, sem1)
    cp2 = pltpu.async_copy(dinv_hbm.at[cidx], dvc, sem2)
    cp1.wait()
    cp2.wait()

    @pl.loop(0, 8)
    def _(k):
      lanes = _lane_iota()
      rows = lanes + k * 16
      a = plsc.load_gather(dvr, [rows, lanes])
      b = plsc.load_gather(dvc, [rows, lanes])
      nbuf[pl.ds(k * 16, 16)] = a * ewv[pl.ds(k * 16, 16)] * b

    pltpu.sync_copy(nbuf, out_hbm.at[pl.ds(off, 128)])


_ACH = 64
_ANCH = EW_PER_TILE // _ACH


@functools.partial(
    pl.kernel, mesh=_MESH,
    compiler_params=pltpu.CompilerParams(needs_layout_passes=False, use_tc_tiling_on_sc=False),
    out_type=jax.ShapeDtypeStruct((NC, NP, D), _f32),
    scratch_types=[
        pltpu.VMEM((_ACH,), _i32), pltpu.VMEM((_ACH,), _i32),
        pltpu.VMEM((_ACH,), _i32), pltpu.VMEM((_ACH,), _i32),
        pltpu.VMEM((_ACH,), _f32), pltpu.VMEM((_ACH,), _f32),
        pltpu.VMEM((_ACH, D), _f32), pltpu.VMEM((_ACH, D), _f32),
        pltpu.VMEM((_ACH, D), _f32), pltpu.VMEM((_ACH, D), _f32),
        pltpu.VMEM_SHARED((NP, D), _f32),
        pltpu.SemaphoreType.DMA, pltpu.SemaphoreType.DMA,
        pltpu.SemaphoreType.DMA, pltpu.SemaphoreType.DMA,
    ],
)
def _sc_appnp(row_hbm, col_hbm, nrm_hbm, x_hbm, zeros_hbm, out_hbm,
              ridx0, ridx1, cidx0, cidx1, nrm0, nrm1,
              rows0, rows1, sbuf0, sbuf1, acc,
              gsem0, gsem1, ssem0, ssem1):
  c, s, base = _core_ids()
  _zero_acc(zeros_hbm, acc, s)

  bufs = ((ridx0, cidx0, nrm0, rows0, sbuf0, gsem0, ssem0),
          (ridx1, cidx1, nrm1, rows1, sbuf1, gsem1, ssem1))

  def load_and_fire(j, b):
    ridx, _, nrm, rows, _, gsem, _ = bufs[b]
    off = base + j * _ACH
    pltpu.sync_copy(row_hbm.at[pl.ds(off, _ACH)], ridx)
    pltpu.sync_copy(nrm_hbm.at[pl.ds(off, _ACH)], nrm)
    pltpu.async_copy(x_hbm.at[ridx], rows, gsem)

  def process(t, j, b, has_next):
    ridx, cidx, nrm, rows, sbuf, gsem, ssem = bufs[b]
    pltpu.make_async_copy(x_hbm.at[ridx], rows, gsem).wait()

    @plsc.parallel_loop(0, _ACH, unroll=4)
    def _(e):
      w = plsc.load_gather(nrm, [_splat(e)])
      for k in range(D // 16):
        sbuf[e, pl.ds(k * 16, 16)] = w * rows[e, pl.ds(k * 16, 16)]

    off = base + j * _ACH
    pltpu.sync_copy(col_hbm.at[pl.ds(off, _ACH)], cidx)

    @pl.when(has_next)
    def _():
      load_and_fire(j + 2, b)

    pltpu.sync_copy(sbuf, acc.at[cidx], add=True)

  load_and_fire(0, 0)
  load_and_fire(1, 1)

  @pl.loop(0, _ANCH // 2)
  def _(t):
    process(t, 2 * t, 0, 2 * t + 2 < _ANCH)
    process(t, 2 * t + 1, 1, 2 * t + 3 < _ANCH)

  _acc_out(acc, out_hbm, c, s)


@functools.partial(
    pl.kernel, mesh=_MESH,
    compiler_params=pltpu.CompilerParams(needs_layout_passes=False, use_tc_tiling_on_sc=False),
    out_type=(jax.ShapeDtypeStruct((EPAD, HP), _f32),
              jax.ShapeDtypeStruct((NC, NP, HP), _f32)),
    scratch_types=[
        pltpu.VMEM((128,), _i32), pltpu.VMEM((128,), _i32),
        pltpu.VMEM((128,), _i32), pltpu.VMEM((128,), _i32),
        pltpu.VMEM((128,), _i32), pltpu.VMEM((128,), _i32),
        pltpu.VMEM((128, HP), _f32), pltpu.VMEM((128, HP), _f32),
        pltpu.VMEM((128, HP), _f32), pltpu.VMEM((128, HP), _f32),
        pltpu.VMEM((128, HP), _f32), pltpu.VMEM((128, HP), _f32),
        pltpu.VMEM((128, HP), _f32), pltpu.VMEM((128, HP), _f32),
        pltpu.VMEM_SHARED((NP, HP), _f32),
        pltpu.SemaphoreType.DMA, pltpu.SemaphoreType.DMA,
        pltpu.SemaphoreType.DMA, pltpu.SemaphoreType.DMA,
    ],
)
def _sc_gat_a(row_hbm, col_hbm, ae_hbm, asrc_hbm, adst_hbm, zeros_hbm,
              s_hbm, den_hbm,
              ridx0, ridx1, cidx0, cidx1, cidxs0, cidxs1,
              aev0, aev1, av0, av1, bv0, bv1, sbuf0, sbuf1,
              acc, gsem0, gsem1, osem0, osem1):
  c, s, base = _core_ids()
  _zero_acc(zeros_hbm, acc, s)

  bufs = ((ridx0, cidx0, cidxs0, aev0, av0, bv0, sbuf0, gsem0, osem0),
          (ridx1, cidx1, cidxs1, aev1, av1, bv1, sbuf1, gsem1, osem1))

  def load_and_fire(j, b):
    ridx, cidx, _, aev, av, bv, _, gsem, _ = bufs[b]
    off = base + j * 128
    pltpu.sync_copy(row_hbm.at[pl.ds(off, 128)], ridx)
    pltpu.sync_copy(col_hbm.at[pl.ds(off, 128)], cidx)
    pltpu.async_copy(ae_hbm.at[pl.ds(off, 128)], aev, gsem)
    pltpu.async_copy(asrc_hbm.at[ridx], av, gsem)
    pltpu.async_copy(adst_hbm.at[cidx], bv, gsem)

  def process(t, j, b, has_next):
    ridx, cidx, cidxs, aev, av, bv, sbuf, gsem, osem = bufs[b]
    off = base + j * 128
    pltpu.make_async_copy(ae_hbm.at[pl.ds(off, 128)], aev, gsem).wait()
    pltpu.make_async_copy(asrc_hbm.at[ridx], av, gsem).wait()
    pltpu.make_async_copy(adst_hbm.at[cidx], bv, gsem).wait()

    @plsc.parallel_loop(0, 128, unroll=4)
    def _(e):
      a = av[e, pl.ds(0, 16)] + bv[e, pl.ds(0, 16)] + aev[e, pl.ds(0, 16)]
      a = jnp.maximum(a, 0.2 * a)
      sbuf[e, pl.ds(0, 16)] = jnp.exp(a)

    pltpu.sync_copy(col_hbm.at[pl.ds(off, 128)], cidxs)

    @pl.when(has_next)
    def _():
      load_and_fire(j + 2, b)

    pltpu.sync_copy(sbuf, s_hbm.at[pl.ds(off, 128)])
    pltpu.sync_copy(sbuf, acc.at[cidxs], add=True)

  load_and_fire(0, 0)
  load_and_fire(1, 1)
  nch = EW_PER_TILE // 128

  @pl.loop(0, nch // 2)
  def _(t):
    process(t, 2 * t, 0, 2 * t + 2 < nch)
    process(t, 2 * t + 1, 1, 2 * t + 3 < nch)

  _acc_out(acc, den_hbm, c, s)


@functools.partial(
    pl.kernel, mesh=_MESH,
    compiler_params=pltpu.CompilerParams(needs_layout_passes=False, use_tc_tiling_on_sc=False),
    out_type=jax.ShapeDtypeStruct((NC, NP, HC), _f32),
    scratch_types=[
        pltpu.VMEM((64,), _i32), pltpu.VMEM((64,), _i32),
        pltpu.VMEM((64,), _i32), pltpu.VMEM((64,), _i32),
        pltpu.VMEM((64,), _i32), pltpu.VMEM((64,), _i32),
        pltpu.VMEM((64, HP), _f32), pltpu.VMEM((64, HP), _f32),
        pltpu.VMEM((64, HP), _f32), pltpu.VMEM((64, HP), _f32),
        pltpu.VMEM((64, XSW), _f32), pltpu.VMEM((64, XSW), _f32),
        pltpu.VMEM((64, HP), _f32), pltpu.VMEM((64, HP), _f32),
        pltpu.VMEM((64, HC), _f32), pltpu.VMEM((64, HC), _f32),
        pltpu.VMEM_SHARED((NP, HC), _f32),
        pltpu.SemaphoreType.DMA, pltpu.SemaphoreType.DMA,
        pltpu.SemaphoreType.DMA, pltpu.SemaphoreType.DMA,
    ],
)
def _sc_gat_b(row_hbm, col_hbm, s_hbm, dfin_hbm, xs_hbm, zeros_hbm, out_hbm,
              ridx0, ridx1, cidx0, cidx1, cidxs0, cidxs1,
              sv0, sv1, dv0, dv1, xrows0, xrows1, att0, att1,
              mbuf0, mbuf1, acc, gsem0, gsem1, ssem0, ssem1):
  c, s, base = _core_ids()
  _zero_acc(zeros_hbm, acc, s)

  bufs = ((ridx0, cidx0, cidxs0, sv0, dv0, xrows0, att0, mbuf0, gsem0, ssem0),
          (ridx1, cidx1, cidxs1, sv1, dv1, xrows1, att1, mbuf1, gsem1, ssem1))

  def load_and_fire(j, b):
    ridx, cidx, _, sv, dv, xrows, _, _, gsem, _ = bufs[b]
    off = base + j * 64
    pltpu.sync_copy(row_hbm.at[pl.ds(off, 64)], ridx)
    pltpu.sync_copy(col_hbm.at[pl.ds(off, 64)], cidx)
    pltpu.async_copy(s_hbm.at[pl.ds(off, 64)], sv, gsem)
    pltpu.async_copy(dfin_hbm.at[cidx], dv, gsem)
    pltpu.async_copy(xs_hbm.at[ridx], xrows, gsem)

  def process(t, j, b, has_next):
    ridx, cidx, cidxs, sv, dv, xrows, att, mbuf, gsem, ssem = bufs[b]
    off = base + j * 64
    pltpu.make_async_copy(s_hbm.at[pl.ds(off, 64)], sv, gsem).wait()
    pltpu.make_async_copy(dfin_hbm.at[cidx], dv, gsem).wait()
    pltpu.make_async_copy(xs_hbm.at[ridx], xrows, gsem).wait()

    @plsc.parallel_loop(0, 64, unroll=2)
    def _(e):
      att[e, pl.ds(0, 16)] = sv[e, pl.ds(0, 16)] / dv[e, pl.ds(0, 16)]

    @plsc.parallel_loop(0, 64, unroll=2)
    def _(e):
      m = [jnp.zeros((16,), _f32) for _ in range(HC // 16)]
      for h in range(HEADS):
        w = plsc.load_gather(att, [_splat(e), _splat(h)])
        for k in range(HC // 16):
          m[k] = m[k] + w * xrows[e, pl.ds(h * HC + k * 16, 16)]
      for k in range(HC // 16):
        mbuf[e, pl.ds(k * 16, 16)] = m[k]

    pltpu.sync_copy(col_hbm.at[pl.ds(off, 64)], cidxs)

    @pl.when(has_next)
    def _():
      load_and_fire(j + 2, b)

    pltpu.sync_copy(mbuf, acc.at[cidxs], add=True)

  load_and_fire(0, 0)
  load_and_fire(1, 1)
  nch = EW_PER_TILE // 64

  @pl.loop(0, nch // 2)
  def _(t):
    process(t, 2 * t, 0, 2 * t + 2 < nch)
    process(t, 2 * t + 1, 1, 2 * t + 3 < nch)

  _acc_out(acc, out_hbm, c, s)


def _tc_edge_body(wdict_n, ea_ref, *refs):
  w = {k: r[...] for k, r in zip(wdict_n, refs[:len(wdict_n)])}
  ew_ref, ae1_ref, ae2_ref, easum_ref = refs[len(wdict_n):]
  i = pl.program_id(0)
  ea = ea_ref[...]
  bs = ea.shape[0]
  rid = i * bs + lax.broadcasted_iota(_i32, (bs, 1), 0)
  valid = rid < E

  def mlp2(z, w1, b1, w2, b2, act):
    z1 = act(z[:, 0:1] * w1[0:1, :] + z[:, 1:2] * w1[1:2, :] + b1)
    return jnp.dot(z1, w2, preferred_element_type=_f32) + b2

  ew = jax.nn.sigmoid(
      mlp2(ea, w['ew_w1'], w['ew_b1'], w['ew_w2'], w['ew_b2'],
           jax.nn.sigmoid))
  ew_ref[...] = jnp.where(valid, ew, 0.0)

  for (w1, b1, w2, b2, wef, oref) in (
      (w['ea1_w1'], w['ea1_b1'], w['ea1_w2'], w['ea1_b2'], w['we1f'],
       ae1_ref),
      (w['ea2_w1'], w['ea2_b1'], w['ea2_w2'], w['ea2_b2'], w['we2f'],
       ae2_ref)):
    eax = mlp2(ea, w1, b1, w2, b2, jax.nn.relu)
    ae = jnp.dot(eax, wef, preferred_element_type=_f32)
    ae = jnp.concatenate([ae, jnp.full((bs, HP - HEADS), NEG, _f32)], axis=1)
    oref[...] = jnp.where(valid, ae, NEG)

  @pl.when(i == 0)
  def _():
    easum_ref[...] = jnp.zeros_like(easum_ref)
  easum_ref[...] += jnp.sum(jnp.where(valid, ea, 0.0), axis=0, keepdims=True)


def _tc_edge(ea_pad, wdict):
  names = list(wdict)
  bs = 4096
  grid = EPAD // bs
  wspecs = [pl.BlockSpec(wdict[k].shape,
                         functools.partial(lambda nd, i: (0,) * nd,
                                           wdict[k].ndim))
            for k in names]
  return pl.pallas_call(
      functools.partial(_tc_edge_body, names),
      grid=(grid,),
      in_specs=[pl.BlockSpec((bs, 2), lambda i: (i, 0))] + wspecs,
      out_specs=[
          pl.BlockSpec((bs, 1), lambda i: (i, 0)),
          pl.BlockSpec((bs, HP), lambda i: (i, 0)),
          pl.BlockSpec((bs, HP), lambda i: (i, 0)),
          pl.BlockSpec((1, 2), lambda i: (0, 0)),
      ],
      out_shape=[
          jax.ShapeDtypeStruct((EPAD, 1), _f32),
          jax.ShapeDtypeStruct((EPAD, HP), _f32),
          jax.ShapeDtypeStruct((EPAD, HP), _f32),
          jax.ShapeDtypeStruct((1, 2), _f32),
      ],
  )(ea_pad, *[wdict[k] for k in names])


def _tc_deg_body(d0_ref, d1_ref, dinv_ref, selfw_ref):
  deg = d0_ref[:, 0:1] + d1_ref[:, 0:1] + 1.0
  dinv_ref[...] = jnp.broadcast_to(lax.rsqrt(deg), dinv_ref.shape)
  selfw_ref[...] = jnp.broadcast_to(1.0 / deg, selfw_ref.shape)


def _tc_deg(d0, d1):
  bs = 1024
  return pl.pallas_call(
      _tc_deg_body,
      grid=(NP // bs,),
      in_specs=[pl.BlockSpec((bs, HP), lambda i: (i, 0))] * 2,
      out_specs=[pl.BlockSpec((bs, HP), lambda i: (i, 0))] * 2,
      out_shape=[jax.ShapeDtypeStruct((NP, HP), _f32)] * 2,
  )(d0, d1)


def _tc_appnp_combine_body(final_relu, p0_ref, p1_ref, xin_ref, h0_ref,
                           selfw_ref, out_ref):
  agg = p0_ref[...] + p1_ref[...] + xin_ref[...] * selfw_ref[:, 0:1]
  out = (1.0 - ALPHA) * agg + ALPHA * h0_ref[...]
  if final_relu:
    out = jnp.maximum(out, 0.0)
  out_ref[...] = out


def _tc_appnp_combine(p0, p1, xin, h0, selfw, final_relu):
  bs = 1024
  return pl.pallas_call(
      functools.partial(_tc_appnp_combine_body, final_relu),
      grid=(NP // bs,),
      in_specs=[pl.BlockSpec((bs, D), lambda i: (i, 0))] * 4 +
               [pl.BlockSpec((bs, HP), lambda i: (i, 0))],
      out_specs=pl.BlockSpec((bs, D), lambda i: (i, 0)),
      out_shape=jax.ShapeDtypeStruct((NP, D), _f32),
  )(p0, p1, xin, h0, selfw)


def _tc_gat_node_body(h_ref, ws_ref, wd_ref, as_ref, ad_ref,
                      xs_ref, asrc_ref, adst_ref):
  h = h_ref[...]
  xs = jnp.dot(h, ws_ref[...], preferred_element_type=_f32)
  xd = jnp.dot(h, wd_ref[...], preferred_element_type=_f32)
  xs_ref[...] = xs
  bs = h.shape[0]

  def headsum(v, a):
    cols = [jnp.sum(v[:, hh * HC:(hh + 1) * HC] * a[:, hh * HC:(hh + 1) * HC],
                    axis=1, keepdims=True) for hh in range(HEADS)]
    cols.append(jnp.zeros((bs, HP - HEADS), _f32))
    return jnp.concatenate(cols, axis=1)

  asrc_ref[...] = headsum(xs, as_ref[...])
  adst_ref[...] = headsum(xd, ad_ref[...])


def _tc_gat_node(h, ws, wd, a_s, a_d):
  bs = 1024
  din = h.shape[1]
  return pl.pallas_call(
      _tc_gat_node_body,
      grid=(NP // bs,),
      in_specs=[
          pl.BlockSpec((bs, din), lambda i: (i, 0)),
          pl.BlockSpec((din, XSW), lambda i: (0, 0)),
          pl.BlockSpec((din, XSW), lambda i: (0, 0)),
          pl.BlockSpec((1, XSW), lambda i: (0, 0)),
          pl.BlockSpec((1, XSW), lambda i: (0, 0)),
      ],
      out_specs=[
          pl.BlockSpec((bs, XSW), lambda i: (i, 0)),
          pl.BlockSpec((bs, HP), lambda i: (i, 0)),
          pl.BlockSpec((bs, HP), lambda i: (i, 0)),
      ],
      out_shape=[
          jax.ShapeDtypeStruct((NP, XSW), _f32),
          jax.ShapeDtypeStruct((NP, HP), _f32),
          jax.ShapeDtypeStruct((NP, HP), _f32),
      ],
  )(h, ws, wd, a_s, a_d)


def _tc_denom_body(d0_ref, d1_ref, asrc_ref, adst_ref, easum_ref,
                   w1_ref, b1_ref, w2_ref, b2_ref, wef_ref,
                   dfin_ref, sself_ref):
  la = easum_ref[...] / float(E)
  z1 = jax.nn.relu(la[:, 0:1] * w1_ref[0:1, :] + la[:, 1:2] * w1_ref[1:2, :]
                   + b1_ref[...])
  z2 = jnp.dot(z1, w2_ref[...], preferred_element_type=_f32) + b2_ref[...]
  aes = jnp.dot(z2, wef_ref[...], preferred_element_type=_f32)
  aes = jnp.concatenate(
      [aes, jnp.full((1, HP - HEADS), NEG, _f32)], axis=1)
  a = asrc_ref[...] + adst_ref[...] + aes
  sself = jnp.exp(jnp.maximum(a, 0.2 * a))
  sself_ref[...] = sself
  dfin_ref[...] = d0_ref[...] + d1_ref[...] + sself + 1e-16


def _tc_denom(d0, d1, asrc, adst, easum, w1, b1, w2, b2, wef):
  bs = 1024
  return pl.pallas_call(
      _tc_denom_body,
      grid=(NP // bs,),
      in_specs=[pl.BlockSpec((bs, HP), lambda i: (i, 0))] * 4 + [
          pl.BlockSpec((1, 2), lambda i: (0, 0)),
          pl.BlockSpec((2, 8), lambda i: (0, 0)),
          pl.BlockSpec((1, 8), lambda i: (0, 0)),
          pl.BlockSpec((8, 4), lambda i: (0, 0)),
          pl.BlockSpec((1, 4), lambda i: (0, 0)),
          pl.BlockSpec((4, HEADS), lambda i: (0, 0)),
      ],
      out_specs=[pl.BlockSpec((bs, HP), lambda i: (i, 0))] * 2,
      out_shape=[jax.ShapeDtypeStruct((NP, HP), _f32)] * 2,
  )(d0, d1, asrc, adst, easum, w1, b1, w2, b2, wef)


def _tc_gat_final_body(p0_ref, p1_ref, sself_ref, dfin_ref, xs_ref, b_ref,
                       out_ref):
  acc = p0_ref[...] + p1_ref[...]
  sself = sself_ref[...]
  dfin = dfin_ref[...]
  xs = xs_ref[...]
  for h in range(HEADS):
    w = sself[:, h:h + 1] / dfin[:, h:h + 1]
    acc = acc + w * xs[:, h * HC:(h + 1) * HC]
  out_ref[...] = jnp.maximum(acc * (1.0 / HEADS) + b_ref[...], 0.0)


def _tc_gat_final(p0, p1, sself, dfin, xs, bias):
  bs = 1024
  return pl.pallas_call(
      _tc_gat_final_body,
      grid=(NP // bs,),
      in_specs=[
          pl.BlockSpec((bs, HC), lambda i: (i, 0)),
          pl.BlockSpec((bs, HC), lambda i: (i, 0)),
          pl.BlockSpec((bs, HP), lambda i: (i, 0)),
          pl.BlockSpec((bs, HP), lambda i: (i, 0)),
          pl.BlockSpec((bs, XSW), lambda i: (i, 0)),
          pl.BlockSpec((1, HC), lambda i: (0, 0)),
      ],
      out_specs=pl.BlockSpec((bs, HC), lambda i: (i, 0)),
      out_shape=jax.ShapeDtypeStruct((NP, HC), _f32),
  )(p0, p1, sself, dfin, xs, bias)


def _tc_pool_body(h_ref, batch_ref, lw_ref, lb_ref, out_ref):
  h = h_ref[...]
  b = batch_ref[...]
  ids = lax.broadcasted_iota(_i32, (1, NG), 1)
  oh = (b == ids).astype(_f32)
  sums = lax.dot_general(oh, h, (((0,), (0,)), ((), ())),
                         preferred_element_type=_f32)
  ones = jnp.ones((NP, 1), _f32)
  cnt = lax.dot_general(oh, ones, (((0,), (0,)), ((), ())),
                        preferred_element_type=_f32)
  pooled = jnp.maximum(sums / jnp.maximum(cnt, 1.0), 0.0)
  logits = jnp.dot(pooled, lw_ref[...], preferred_element_type=_f32) \
      + lb_ref[...]
  m = jnp.max(logits, axis=1, keepdims=True)
  sh = logits - m
  out_ref[...] = sh - jnp.log(jnp.sum(jnp.exp(sh), axis=1, keepdims=True))


def _tc_pool(h, batch2d, lw, lb):
  return pl.pallas_call(
      _tc_pool_body,
      grid=(1,),
      in_specs=[
          pl.BlockSpec((NP, HC), lambda i: (0, 0)),
          pl.BlockSpec((NP, 1), lambda i: (0, 0)),
          pl.BlockSpec((HC, NCLS), lambda i: (0, 0)),
          pl.BlockSpec((1, NCLS), lambda i: (0, 0)),
      ],
      out_specs=pl.BlockSpec((NG, NCLS), lambda i: (0, 0)),
      out_shape=jax.ShapeDtypeStruct((NG, NCLS), _f32),
  )(h, batch2d, lw, lb)


def kernel(x, edge_attr, params, edge_index, batch):
  p = params
  row = jnp.pad(edge_index[0], (0, EPAD - E))
  col = jnp.pad(edge_index[1], (0, EPAD - E))
  ea_pad = jnp.pad(edge_attr, ((0, EPAD - E), (0, 0)))

  def fold_we(we, a_e):
    return (we.reshape(4, HEADS, HC) * a_e[0][None]).sum(-1)

  wdict = {
      'ew_w1': p['ew_w1'], 'ew_b1': p['ew_b1'].reshape(1, 4),
      'ew_w2': p['ew_w2'], 'ew_b2': p['ew_b2'].reshape(1, 1),
      'ea1_w1': p['ea1_w1'], 'ea1_b1': p['ea1_b1'].reshape(1, 8),
      'ea1_w2': p['ea1_w2'], 'ea1_b2': p['ea1_b2'].reshape(1, 4),
      'ea2_w1': p['ea2_w1'], 'ea2_b1': p['ea2_b1'].reshape(1, 8),
      'ea2_w2': p['ea2_w2'], 'ea2_b2': p['ea2_b2'].reshape(1, 4),
      'we1f': fold_we(p['g1_we'], p['g1_ae']),
      'we2f': fold_we(p['g2_we'], p['g2_ae']),
  }
  ew2d, ae1, ae2, easum = _tc_edge(ea_pad, wdict)
  ew = ew2d.reshape(EPAD)
  xp = jnp.pad(x, ((0, NP - N), (0, 0)))
  batch_p = jnp.pad(batch.astype(_i32), (0, NP - N), constant_values=NG)

  zeros16 = jnp.zeros((ROWS_PER_TILE, HP), _f32)
  zeros128 = jnp.zeros((ROWS_PER_TILE, D), _f32)
  zeros64 = jnp.zeros((ROWS_PER_TILE, HC), _f32)

  degp = _sc_deg(col, ew, zeros16)
  dinv, selfw = _tc_deg(degp[0], degp[1])
  dinv_flat = dinv
  nrm = _sc_norm(row, col, ew, dinv_flat)

  h0 = xp
  out = xp
  for it in range(KPROP):
    part = _sc_appnp(row, col, nrm, out, zeros128)
    out = _tc_appnp_combine(part[0], part[1], out, h0, selfw,
                            final_relu=(it == KPROP - 1))
  h = out

  for (ws, wd, a_s, a_d, bias, ae, wef_n) in (
      (p['g1_ws'], p['g1_wd'], p['g1_as'], p['g1_ad'], p['g1_b'], ae1,
       ('ea1_w1', 'ea1_b1', 'ea1_w2', 'ea1_b2', 'we1f')),
      (p['g2_ws'], p['g2_wd'], p['g2_as'], p['g2_ad'], p['g2_b'], ae2,
       ('ea2_w1', 'ea2_b1', 'ea2_w2', 'ea2_b2', 'we2f'))):
    xs, asrc, adst = _tc_gat_node(h, ws, wd,
                                  a_s.reshape(1, XSW), a_d.reshape(1, XSW))
    sarr, denp = _sc_gat_a(row, col, ae, asrc, adst, zeros16)
    dfin, sself = _tc_denom(denp[0], denp[1], asrc, adst, easum,
                            wdict[wef_n[0]], wdict[wef_n[1]],
                            wdict[wef_n[2]], wdict[wef_n[3]],
                            wdict[wef_n[4]])
    part = _sc_gat_b(row, col, sarr, dfin, xs, zeros64)
    h = _tc_gat_final(part[0], part[1], sself, dfin, xs,
                      bias.reshape(1, HC))

  return _tc_pool(h, batch_p.reshape(NP, 1),
                  p['lin_w'], p['lin_b'].reshape(1, NCLS))


# speedup vs baseline: 20.9989x; 1.2517x over previous
import functools

import jax
import jax.numpy as jnp
from jax import lax
from jax.experimental import pallas as pl
from jax.experimental.pallas import tpu as pltpu
from jax.experimental.pallas import tpu_sc as plsc

N = 10000
NP = 10240
E = 160000
D = 128
HEADS = 6
HC = 64
NG = 64
NCLS = 10
KPROP = 5
ALPHA = 0.1
HP = 16
XSW = HEADS * HC

NC = 2
NS = 16
NW = NC * NS
EPAD = 163840
EW_PER_TILE = EPAD // NW
ROWS_PER_TILE = NP // NS
NEG = -1e9

_MESH = plsc.VectorSubcoreMesh(core_axis_name="c", subcore_axis_name="s")
_f32 = jnp.float32
_i32 = jnp.int32


def _lane_iota():
  return lax.broadcasted_iota(_i32, (16,), 0)


def _splat(v):
  return jnp.full((16,), v, _i32)


def _core_ids():
  c = lax.axis_index("c")
  s = lax.axis_index("s")
  return c, s, (s * NC + c) * EW_PER_TILE


def _zero_acc(zeros_hbm, acc, s):
  pltpu.sync_copy(zeros_hbm, acc.at[pl.ds(s * ROWS_PER_TILE, ROWS_PER_TILE)])
  plsc.subcore_barrier()


def _acc_out(acc, out, c, s):
  plsc.subcore_barrier()
  r0 = s * ROWS_PER_TILE
  pltpu.sync_copy(acc.at[pl.ds(r0, ROWS_PER_TILE)],
                  out.at[c, pl.ds(r0, ROWS_PER_TILE)])


@functools.partial(
    pl.kernel, mesh=_MESH,
    compiler_params=pltpu.CompilerParams(needs_layout_passes=False, use_tc_tiling_on_sc=False),
    out_type=jax.ShapeDtypeStruct((NC, NP, HP), _f32),
    scratch_types=[
        pltpu.VMEM((128,), _i32),
        pltpu.VMEM((128,), _f32),
        pltpu.VMEM((128, HP), _f32),
        pltpu.VMEM_SHARED((NP, HP), _f32),
    ],
)
def _sc_deg(col_hbm, ew_hbm, zeros_hbm, out_hbm, cidx, ewv, sbuf, acc):
  c, s, base = _core_ids()
  _zero_acc(zeros_hbm, acc, s)

  @pl.loop(0, EW_PER_TILE // 128)
  def _(j):
    off = base + j * 128
    pltpu.sync_copy(col_hbm.at[pl.ds(off, 128)], cidx)
    pltpu.sync_copy(ew_hbm.at[pl.ds(off, 128)], ewv)

    @pl.loop(0, 128)
    def _(e):
      w = plsc.load_gather(ewv, [_splat(e)])
      unit = jnp.where(_lane_iota() == 0, 1.0, 0.0).astype(_f32)
      sbuf[e, pl.ds(0, HP)] = w * unit

    pltpu.sync_copy(sbuf, acc.at[cidx], add=True)

  _acc_out(acc, out_hbm, c, s)


@functools.partial(
    pl.kernel, mesh=_MESH,
    compiler_params=pltpu.CompilerParams(needs_layout_passes=False, use_tc_tiling_on_sc=False),
    out_type=jax.ShapeDtypeStruct((EPAD,), _f32),
    scratch_types=[
        pltpu.VMEM((128,), _i32),
        pltpu.VMEM((128,), _i32),
        pltpu.VMEM((128,), _f32),
        pltpu.VMEM((128, HP), _f32),
        pltpu.VMEM((128, HP), _f32),
        pltpu.VMEM((128,), _f32),
        pltpu.SemaphoreType.DMA,
        pltpu.SemaphoreType.DMA,
    ],
)
def _sc_norm(row_hbm, col_hbm, ew_hbm, dinv_hbm, out_hbm,
             ridx, cidx, ewv, dvr, dvc, nbuf, sem1, sem2):
  _, _, base = _core_ids()

  @pl.loop(0, EW_PER_TILE // 128)
  def _(j):
    off = base + j * 128
    pltpu.sync_copy(row_hbm.at[pl.ds(off, 128)], ridx)
    pltpu.sync_copy(col_hbm.at[pl.ds(off, 128)], cidx)
    pltpu.sync_copy(ew_hbm.at[pl.ds(off, 128)], ewv)
    cp1 = pltpu.async_copy(dinv_hbm.at[ridx], dvr, sem1)
    cp2 = pltpu.async_copy(dinv_hbm.at[cidx], dvc, sem2)
    cp1.wait()
    cp2.wait()

    @pl.loop(0, 8)
    def _(k):
      lanes = _lane_iota()
      rows = lanes + k * 16
      a = plsc.load_gather(dvr, [rows, lanes])
      b = plsc.load_gather(dvc, [rows, lanes])
      nbuf[pl.ds(k * 16, 16)] = a * ewv[pl.ds(k * 16, 16)] * b

    pltpu.sync_copy(nbuf, out_hbm.at[pl.ds(off, 128)])


_ACH = 128
_ANCH = EW_PER_TILE // _ACH
HD = D // 2


@functools.partial(
    pl.kernel, mesh=_MESH,
    compiler_params=pltpu.CompilerParams(needs_layout_passes=False, use_tc_tiling_on_sc=False),
    out_type=jax.ShapeDtypeStruct((NC, NP, D), _f32),
    scratch_types=[
        pltpu.VMEM((EW_PER_TILE,), _i32),
        pltpu.VMEM((EW_PER_TILE,), _i32),
        pltpu.VMEM((EW_PER_TILE,), _f32),
        pltpu.VMEM((_ACH, HD), _f32),
        pltpu.VMEM((_ACH, HD), _f32),
        pltpu.VMEM_SHARED((NP, HD), _f32),
        pltpu.VMEM_SHARED((NP, HD), _f32),
    ],
)
def _sc_appnp(row_hbm, col_hbm, nrm_hbm, xlo_hbm, xhi_hbm, zeros_hbm, out_hbm,
              ridx_all, cidx_all, nrm_all, rows, sbuf, x_sh, acc):
  c, s, base = _core_ids()
  r0 = s * ROWS_PER_TILE
  pltpu.sync_copy(row_hbm.at[pl.ds(base, EW_PER_TILE)], ridx_all)
  pltpu.sync_copy(col_hbm.at[pl.ds(base, EW_PER_TILE)], cidx_all)
  pltpu.sync_copy(nrm_hbm.at[pl.ds(base, EW_PER_TILE)], nrm_all)

  for p, x_hbm in enumerate((xlo_hbm, xhi_hbm)):
    pltpu.sync_copy(x_hbm.at[pl.ds(r0, ROWS_PER_TILE)],
                    x_sh.at[pl.ds(r0, ROWS_PER_TILE)])
    pltpu.sync_copy(zeros_hbm, acc.at[pl.ds(r0, ROWS_PER_TILE)])
    plsc.subcore_barrier()

    @pl.loop(0, _ANCH)
    def _(j):
      off = j * _ACH
      pltpu.sync_copy(x_sh.at[ridx_all.at[pl.ds(off, _ACH)]], rows)

      @plsc.parallel_loop(0, _ACH, unroll=4)
      def _(e):
        w = plsc.load_gather(nrm_all, [_splat(off + e)])
        for k in range(HD // 16):
          sbuf[e, pl.ds(k * 16, 16)] = w * rows[e, pl.ds(k * 16, 16)]

      pltpu.sync_copy(sbuf, acc.at[cidx_all.at[pl.ds(off, _ACH)]], add=True)

    plsc.subcore_barrier()
    pltpu.sync_copy(acc.at[pl.ds(r0, ROWS_PER_TILE)],
                    out_hbm.at[c, pl.ds(r0, ROWS_PER_TILE), pl.ds(p * HD, HD)])


@functools.partial(
    pl.kernel, mesh=_MESH,
    compiler_params=pltpu.CompilerParams(needs_layout_passes=False, use_tc_tiling_on_sc=False),
    out_type=(jax.ShapeDtypeStruct((EPAD, HP), _f32),
              jax.ShapeDtypeStruct((NC, NP, HP), _f32)),
    scratch_types=[
        pltpu.VMEM((128,), _i32), pltpu.VMEM((128,), _i32),
        pltpu.VMEM((128,), _i32), pltpu.VMEM((128,), _i32),
        pltpu.VMEM((128,), _i32), pltpu.VMEM((128,), _i32),
        pltpu.VMEM((128, HP), _f32), pltpu.VMEM((128, HP), _f32),
        pltpu.VMEM((128, HP), _f32), pltpu.VMEM((128, HP), _f32),
        pltpu.VMEM((128, HP), _f32), pltpu.VMEM((128, HP), _f32),
        pltpu.VMEM((128, HP), _f32), pltpu.VMEM((128, HP), _f32),
        pltpu.VMEM_SHARED((NP, HP), _f32),
        pltpu.SemaphoreType.DMA, pltpu.SemaphoreType.DMA,
        pltpu.SemaphoreType.DMA, pltpu.SemaphoreType.DMA,
    ],
)
def _sc_gat_a(row_hbm, col_hbm, ae_hbm, asrc_hbm, adst_hbm, zeros_hbm,
              s_hbm, den_hbm,
              ridx0, ridx1, cidx0, cidx1, cidxs0, cidxs1,
              aev0, aev1, av0, av1, bv0, bv1, sbuf0, sbuf1,
              acc, gsem0, gsem1, osem0, osem1):
  c, s, base = _core_ids()
  _zero_acc(zeros_hbm, acc, s)

  bufs = ((ridx0, cidx0, cidxs0, aev0, av0, bv0, sbuf0, gsem0, osem0),
          (ridx1, cidx1, cidxs1, aev1, av1, bv1, sbuf1, gsem1, osem1))

  def load_and_fire(j, b):
    ridx, cidx, _, aev, av, bv, _, gsem, _ = bufs[b]
    off = base + j * 128
    pltpu.sync_copy(row_hbm.at[pl.ds(off, 128)], ridx)
    pltpu.sync_copy(col_hbm.at[pl.ds(off, 128)], cidx)
    pltpu.async_copy(ae_hbm.at[pl.ds(off, 128)], aev, gsem)
    pltpu.async_copy(asrc_hbm.at[ridx], av, gsem)
    pltpu.async_copy(adst_hbm.at[cidx], bv, gsem)

  def process(t, j, b, has_next):
    ridx, cidx, cidxs, aev, av, bv, sbuf, gsem, osem = bufs[b]
    off = base + j * 128
    pltpu.make_async_copy(ae_hbm.at[pl.ds(off, 128)], aev, gsem).wait()
    pltpu.make_async_copy(asrc_hbm.at[ridx], av, gsem).wait()
    pltpu.make_async_copy(adst_hbm.at[cidx], bv, gsem).wait()

    @plsc.parallel_loop(0, 128, unroll=4)
    def _(e):
      a = av[e, pl.ds(0, 16)] + bv[e, pl.ds(0, 16)] + aev[e, pl.ds(0, 16)]
      a = jnp.maximum(a, 0.2 * a)
      sbuf[e, pl.ds(0, 16)] = jnp.exp(a)

    pltpu.sync_copy(col_hbm.at[pl.ds(off, 128)], cidxs)

    @pl.when(has_next)
    def _():
      load_and_fire(j + 2, b)

    pltpu.sync_copy(sbuf, s_hbm.at[pl.ds(off, 128)])
    pltpu.sync_copy(sbuf, acc.at[cidxs], add=True)

  load_and_fire(0, 0)
  load_and_fire(1, 1)
  nch = EW_PER_TILE // 128

  @pl.loop(0, nch // 2)
  def _(t):
    process(t, 2 * t, 0, 2 * t + 2 < nch)
    process(t, 2 * t + 1, 1, 2 * t + 3 < nch)

  _acc_out(acc, den_hbm, c, s)


@functools.partial(
    pl.kernel, mesh=_MESH,
    compiler_params=pltpu.CompilerParams(needs_layout_passes=False, use_tc_tiling_on_sc=False),
    out_type=jax.ShapeDtypeStruct((NC, NP, HC), _f32),
    scratch_types=[
        pltpu.VMEM((64,), _i32), pltpu.VMEM((64,), _i32),
        pltpu.VMEM((64,), _i32), pltpu.VMEM((64,), _i32),
        pltpu.VMEM((64,), _i32), pltpu.VMEM((64,), _i32),
        pltpu.VMEM((64, HP), _f32), pltpu.VMEM((64, HP), _f32),
        pltpu.VMEM((64, HP), _f32), pltpu.VMEM((64, HP), _f32),
        pltpu.VMEM((64, XSW), _f32), pltpu.VMEM((64, XSW), _f32),
        pltpu.VMEM((64, HP), _f32), pltpu.VMEM((64, HP), _f32),
        pltpu.VMEM((64, HC), _f32), pltpu.VMEM((64, HC), _f32),
        pltpu.VMEM_SHARED((NP, HC), _f32),
        pltpu.SemaphoreType.DMA, pltpu.SemaphoreType.DMA,
        pltpu.SemaphoreType.DMA, pltpu.SemaphoreType.DMA,
    ],
)
def _sc_gat_b(row_hbm, col_hbm, s_hbm, dfin_hbm, xs_hbm, zeros_hbm, out_hbm,
              ridx0, ridx1, cidx0, cidx1, cidxs0, cidxs1,
              sv0, sv1, dv0, dv1, xrows0, xrows1, att0, att1,
              mbuf0, mbuf1, acc, gsem0, gsem1, ssem0, ssem1):
  c, s, base = _core_ids()
  _zero_acc(zeros_hbm, acc, s)

  bufs = ((ridx0, cidx0, cidxs0, sv0, dv0, xrows0, att0, mbuf0, gsem0, ssem0),
          (ridx1, cidx1, cidxs1, sv1, dv1, xrows1, att1, mbuf1, gsem1, ssem1))

  def load_and_fire(j, b):
    ridx, cidx, _, sv, dv, xrows, _, _, gsem, _ = bufs[b]
    off = base + j * 64
    pltpu.sync_copy(row_hbm.at[pl.ds(off, 64)], ridx)
    pltpu.sync_copy(col_hbm.at[pl.ds(off, 64)], cidx)
    pltpu.async_copy(s_hbm.at[pl.ds(off, 64)], sv, gsem)
    pltpu.async_copy(dfin_hbm.at[cidx], dv, gsem)
    pltpu.async_copy(xs_hbm.at[ridx], xrows, gsem)

  def process(t, j, b, has_next):
    ridx, cidx, cidxs, sv, dv, xrows, att, mbuf, gsem, ssem = bufs[b]
    off = base + j * 64
    pltpu.make_async_copy(s_hbm.at[pl.ds(off, 64)], sv, gsem).wait()
    pltpu.make_async_copy(dfin_hbm.at[cidx], dv, gsem).wait()
    pltpu.make_async_copy(xs_hbm.at[ridx], xrows, gsem).wait()

    @plsc.parallel_loop(0, 64, unroll=2)
    def _(e):
      att[e, pl.ds(0, 16)] = sv[e, pl.ds(0, 16)] / dv[e, pl.ds(0, 16)]

    @plsc.parallel_loop(0, 64, unroll=2)
    def _(e):
      m = [jnp.zeros((16,), _f32) for _ in range(HC // 16)]
      for h in range(HEADS):
        w = plsc.load_gather(att, [_splat(e), _splat(h)])
        for k in range(HC // 16):
          m[k] = m[k] + w * xrows[e, pl.ds(h * HC + k * 16, 16)]
      for k in range(HC // 16):
        mbuf[e, pl.ds(k * 16, 16)] = m[k]

    pltpu.sync_copy(col_hbm.at[pl.ds(off, 64)], cidxs)

    @pl.when(has_next)
    def _():
      load_and_fire(j + 2, b)

    pltpu.sync_copy(mbuf, acc.at[cidxs], add=True)

  load_and_fire(0, 0)
  load_and_fire(1, 1)
  nch = EW_PER_TILE // 64

  @pl.loop(0, nch // 2)
  def _(t):
    process(t, 2 * t, 0, 2 * t + 2 < nch)
    process(t, 2 * t + 1, 1, 2 * t + 3 < nch)

  _acc_out(acc, out_hbm, c, s)


def _tc_edge_body(wdict_n, ea_ref, *refs):
  w = {k: r[...] for k, r in zip(wdict_n, refs[:len(wdict_n)])}
  ew_ref, ae1_ref, ae2_ref, easum_ref = refs[len(wdict_n):]
  i = pl.program_id(0)
  ea = ea_ref[...]
  bs = ea.shape[0]
  rid = i * bs + lax.broadcasted_iota(_i32, (bs, 1), 0)
  valid = rid < E

  def mlp2(z, w1, b1, w2, b2, act):
    z1 = act(z[:, 0:1] * w1[0:1, :] + z[:, 1:2] * w1[1:2, :] + b1)
    return jnp.dot(z1, w2, preferred_element_type=_f32) + b2

  ew = jax.nn.sigmoid(
      mlp2(ea, w['ew_w1'], w['ew_b1'], w['ew_w2'], w['ew_b2'],
           jax.nn.sigmoid))
  ew_ref[...] = jnp.where(valid, ew, 0.0)

  for (w1, b1, w2, b2, wef, oref) in (
      (w['ea1_w1'], w['ea1_b1'], w['ea1_w2'], w['ea1_b2'], w['we1f'],
       ae1_ref),
      (w['ea2_w1'], w['ea2_b1'], w['ea2_w2'], w['ea2_b2'], w['we2f'],
       ae2_ref)):
    eax = mlp2(ea, w1, b1, w2, b2, jax.nn.relu)
    ae = jnp.dot(eax, wef, preferred_element_type=_f32)
    ae = jnp.concatenate([ae, jnp.full((bs, HP - HEADS), NEG, _f32)], axis=1)
    oref[...] = jnp.where(valid, ae, NEG)

  @pl.when(i == 0)
  def _():
    easum_ref[...] = jnp.zeros_like(easum_ref)
  easum_ref[...] += jnp.sum(jnp.where(valid, ea, 0.0), axis=0, keepdims=True)


def _tc_edge(ea_pad, wdict):
  names = list(wdict)
  bs = 4096
  grid = EPAD // bs
  wspecs = [pl.BlockSpec(wdict[k].shape,
                         functools.partial(lambda nd, i: (0,) * nd,
                                           wdict[k].ndim))
            for k in names]
  return pl.pallas_call(
      functools.partial(_tc_edge_body, names),
      grid=(grid,),
      in_specs=[pl.BlockSpec((bs, 2), lambda i: (i, 0))] + wspecs,
      out_specs=[
          pl.BlockSpec((bs, 1), lambda i: (i, 0)),
          pl.BlockSpec((bs, HP), lambda i: (i, 0)),
          pl.BlockSpec((bs, HP), lambda i: (i, 0)),
          pl.BlockSpec((1, 2), lambda i: (0, 0)),
      ],
      out_shape=[
          jax.ShapeDtypeStruct((EPAD, 1), _f32),
          jax.ShapeDtypeStruct((EPAD, HP), _f32),
          jax.ShapeDtypeStruct((EPAD, HP), _f32),
          jax.ShapeDtypeStruct((1, 2), _f32),
      ],
  )(ea_pad, *[wdict[k] for k in names])


def _tc_deg_body(d0_ref, d1_ref, dinv_ref, selfw_ref):
  deg = d0_ref[:, 0:1] + d1_ref[:, 0:1] + 1.0
  dinv_ref[...] = jnp.broadcast_to(lax.rsqrt(deg), dinv_ref.shape)
  selfw_ref[...] = jnp.broadcast_to(1.0 / deg, selfw_ref.shape)


def _tc_deg(d0, d1):
  bs = 1024
  return pl.pallas_call(
      _tc_deg_body,
      grid=(NP // bs,),
      in_specs=[pl.BlockSpec((bs, HP), lambda i: (i, 0))] * 2,
      out_specs=[pl.BlockSpec((bs, HP), lambda i: (i, 0))] * 2,
      out_shape=[jax.ShapeDtypeStruct((NP, HP), _f32)] * 2,
  )(d0, d1)


def _tc_appnp_combine_body(final_relu, p0_ref, p1_ref, xin_ref, h0_ref,
                           selfw_ref, out_ref):
  agg = p0_ref[...] + p1_ref[...] + xin_ref[...] * selfw_ref[:, 0:1]
  out = (1.0 - ALPHA) * agg + ALPHA * h0_ref[...]
  if final_relu:
    out = jnp.maximum(out, 0.0)
  out_ref[...] = out


def _tc_appnp_combine(p0, p1, xin, h0, selfw, final_relu):
  bs = 1024
  return pl.pallas_call(
      functools.partial(_tc_appnp_combine_body, final_relu),
      grid=(NP // bs,),
      in_specs=[pl.BlockSpec((bs, D), lambda i: (i, 0))] * 4 +
               [pl.BlockSpec((bs, HP), lambda i: (i, 0))],
      out_specs=pl.BlockSpec((bs, D), lambda i: (i, 0)),
      out_shape=jax.ShapeDtypeStruct((NP, D), _f32),
  )(p0, p1, xin, h0, selfw)


def _tc_gat_node_body(h_ref, ws_ref, wd_ref, as_ref, ad_ref,
                      xs_ref, asrc_ref, adst_ref):
  h = h_ref[...]
  xs = jnp.dot(h, ws_ref[...], preferred_element_type=_f32)
  xd = jnp.dot(h, wd_ref[...], preferred_element_type=_f32)
  xs_ref[...] = xs
  bs = h.shape[0]

  def headsum(v, a):
    cols = [jnp.sum(v[:, hh * HC:(hh + 1) * HC] * a[:, hh * HC:(hh + 1) * HC],
                    axis=1, keepdims=True) for hh in range(HEADS)]
    cols.append(jnp.zeros((bs, HP - HEADS), _f32))
    return jnp.concatenate(cols, axis=1)

  asrc_ref[...] = headsum(xs, as_ref[...])
  adst_ref[...] = headsum(xd, ad_ref[...])


def _tc_gat_node(h, ws, wd, a_s, a_d):
  bs = 1024
  din = h.shape[1]
  return pl.pallas_call(
      _tc_gat_node_body,
      grid=(NP // bs,),
      in_specs=[
          pl.BlockSpec((bs, din), lambda i: (i, 0)),
          pl.BlockSpec((din, XSW), lambda i: (0, 0)),
          pl.BlockSpec((din, XSW), lambda i: (0, 0)),
          pl.BlockSpec((1, XSW), lambda i: (0, 0)),
          pl.BlockSpec((1, XSW), lambda i: (0, 0)),
      ],
      out_specs=[
          pl.BlockSpec((bs, XSW), lambda i: (i, 0)),
          pl.BlockSpec((bs, HP), lambda i: (i, 0)),
          pl.BlockSpec((bs, HP), lambda i: (i, 0)),
      ],
      out_shape=[
          jax.ShapeDtypeStruct((NP, XSW), _f32),
          jax.ShapeDtypeStruct((NP, HP), _f32),
          jax.ShapeDtypeStruct((NP, HP), _f32),
      ],
  )(h, ws, wd, a_s, a_d)


def _tc_denom_body(d0_ref, d1_ref, asrc_ref, adst_ref, easum_ref,
                   w1_ref, b1_ref, w2_ref, b2_ref, wef_ref,
                   dfin_ref, sself_ref):
  la = easum_ref[...] / float(E)
  z1 = jax.nn.relu(la[:, 0:1] * w1_ref[0:1, :] + la[:, 1:2] * w1_ref[1:2, :]
                   + b1_ref[...])
  z2 = jnp.dot(z1, w2_ref[...], preferred_element_type=_f32) + b2_ref[...]
  aes = jnp.dot(z2, wef_ref[...], preferred_element_type=_f32)
  aes = jnp.concatenate(
      [aes, jnp.full((1, HP - HEADS), NEG, _f32)], axis=1)
  a = asrc_ref[...] + adst_ref[...] + aes
  sself = jnp.exp(jnp.maximum(a, 0.2 * a))
  sself_ref[...] = sself
  dfin_ref[...] = d0_ref[...] + d1_ref[...] + sself + 1e-16


def _tc_denom(d0, d1, asrc, adst, easum, w1, b1, w2, b2, wef):
  bs = 1024
  return pl.pallas_call(
      _tc_denom_body,
      grid=(NP // bs,),
      in_specs=[pl.BlockSpec((bs, HP), lambda i: (i, 0))] * 4 + [
          pl.BlockSpec((1, 2), lambda i: (0, 0)),
          pl.BlockSpec((2, 8), lambda i: (0, 0)),
          pl.BlockSpec((1, 8), lambda i: (0, 0)),
          pl.BlockSpec((8, 4), lambda i: (0, 0)),
          pl.BlockSpec((1, 4), lambda i: (0, 0)),
          pl.BlockSpec((4, HEADS), lambda i: (0, 0)),
      ],
      out_specs=[pl.BlockSpec((bs, HP), lambda i: (i, 0))] * 2,
      out_shape=[jax.ShapeDtypeStruct((NP, HP), _f32)] * 2,
  )(d0, d1, asrc, adst, easum, w1, b1, w2, b2, wef)


def _tc_gat_final_body(p0_ref, p1_ref, sself_ref, dfin_ref, xs_ref, b_ref,
                       out_ref):
  acc = p0_ref[...] + p1_ref[...]
  sself = sself_ref[...]
  dfin = dfin_ref[...]
  xs = xs_ref[...]
  for h in range(HEADS):
    w = sself[:, h:h + 1] / dfin[:, h:h + 1]
    acc = acc + w * xs[:, h * HC:(h + 1) * HC]
  out_ref[...] = jnp.maximum(acc * (1.0 / HEADS) + b_ref[...], 0.0)


def _tc_gat_final(p0, p1, sself, dfin, xs, bias):
  bs = 1024
  return pl.pallas_call(
      _tc_gat_final_body,
      grid=(NP // bs,),
      in_specs=[
          pl.BlockSpec((bs, HC), lambda i: (i, 0)),
          pl.BlockSpec((bs, HC), lambda i: (i, 0)),
          pl.BlockSpec((bs, HP), lambda i: (i, 0)),
          pl.BlockSpec((bs, HP), lambda i: (i, 0)),
          pl.BlockSpec((bs, XSW), lambda i: (i, 0)),
          pl.BlockSpec((1, HC), lambda i: (0, 0)),
      ],
      out_specs=pl.BlockSpec((bs, HC), lambda i: (i, 0)),
      out_shape=jax.ShapeDtypeStruct((NP, HC), _f32),
  )(p0, p1, sself, dfin, xs, bias)


def _tc_pool_body(h_ref, batch_ref, lw_ref, lb_ref, out_ref):
  h = h_ref[...]
  b = batch_ref[...]
  ids = lax.broadcasted_iota(_i32, (1, NG), 1)
  oh = (b == ids).astype(_f32)
  sums = lax.dot_general(oh, h, (((0,), (0,)), ((), ())),
                         preferred_element_type=_f32)
  ones = jnp.ones((NP, 1), _f32)
  cnt = lax.dot_general(oh, ones, (((0,), (0,)), ((), ())),
                        preferred_element_type=_f32)
  pooled = jnp.maximum(sums / jnp.maximum(cnt, 1.0), 0.0)
  logits = jnp.dot(pooled, lw_ref[...], preferred_element_type=_f32) \
      + lb_ref[...]
  m = jnp.max(logits, axis=1, keepdims=True)
  sh = logits - m
  out_ref[...] = sh - jnp.log(jnp.sum(jnp.exp(sh), axis=1, keepdims=True))


def _tc_pool(h, batch2d, lw, lb):
  return pl.pallas_call(
      _tc_pool_body,
      grid=(1,),
      in_specs=[
          pl.BlockSpec((NP, HC), lambda i: (0, 0)),
          pl.BlockSpec((NP, 1), lambda i: (0, 0)),
          pl.BlockSpec((HC, NCLS), lambda i: (0, 0)),
          pl.BlockSpec((1, NCLS), lambda i: (0, 0)),
      ],
      out_specs=pl.BlockSpec((NG, NCLS), lambda i: (0, 0)),
      out_shape=jax.ShapeDtypeStruct((NG, NCLS), _f32),
  )(h, batch2d, lw, lb)


def kernel(x, edge_attr, params, edge_index, batch):
  p = params
  row = jnp.pad(edge_index[0], (0, EPAD - E))
  col = jnp.pad(edge_index[1], (0, EPAD - E))
  ea_pad = jnp.pad(edge_attr, ((0, EPAD - E), (0, 0)))

  def fold_we(we, a_e):
    return (we.reshape(4, HEADS, HC) * a_e[0][None]).sum(-1)

  wdict = {
      'ew_w1': p['ew_w1'], 'ew_b1': p['ew_b1'].reshape(1, 4),
      'ew_w2': p['ew_w2'], 'ew_b2': p['ew_b2'].reshape(1, 1),
      'ea1_w1': p['ea1_w1'], 'ea1_b1': p['ea1_b1'].reshape(1, 8),
      'ea1_w2': p['ea1_w2'], 'ea1_b2': p['ea1_b2'].reshape(1, 4),
      'ea2_w1': p['ea2_w1'], 'ea2_b1': p['ea2_b1'].reshape(1, 8),
      'ea2_w2': p['ea2_w2'], 'ea2_b2': p['ea2_b2'].reshape(1, 4),
      'we1f': fold_we(p['g1_we'], p['g1_ae']),
      'we2f': fold_we(p['g2_we'], p['g2_ae']),
  }
  ew2d, ae1, ae2, easum = _tc_edge(ea_pad, wdict)
  ew = ew2d.reshape(EPAD)
  xp = jnp.pad(x, ((0, NP - N), (0, 0)))
  batch_p = jnp.pad(batch.astype(_i32), (0, NP - N), constant_values=NG)

  zeros16 = jnp.zeros((ROWS_PER_TILE, HP), _f32)
  zeros64 = jnp.zeros((ROWS_PER_TILE, HC), _f32)

  degp = _sc_deg(col, ew, zeros16)
  dinv, selfw = _tc_deg(degp[0], degp[1])
  dinv_flat = dinv
  nrm = _sc_norm(row, col, ew, dinv_flat)

  h0 = xp
  out = xp
  for it in range(KPROP):
    part = _sc_appnp(row, col, nrm, out[:, :HD], out[:, HD:], zeros64)
    out = _tc_appnp_combine(part[0], part[1], out, h0, selfw,
                            final_relu=(it == KPROP - 1))
  h = out

  for (ws, wd, a_s, a_d, bias, ae, wef_n) in (
      (p['g1_ws'], p['g1_wd'], p['g1_as'], p['g1_ad'], p['g1_b'], ae1,
       ('ea1_w1', 'ea1_b1', 'ea1_w2', 'ea1_b2', 'we1f')),
      (p['g2_ws'], p['g2_wd'], p['g2_as'], p['g2_ad'], p['g2_b'], ae2,
       ('ea2_w1', 'ea2_b1', 'ea2_w2', 'ea2_b2', 'we2f'))):
    xs, asrc, adst = _tc_gat_node(h, ws, wd,
                                  a_s.reshape(1, XSW), a_d.reshape(1, XSW))
    sarr, denp = _sc_gat_a(row, col, ae, asrc, adst, zeros16)
    dfin, sself = _tc_denom(denp[0], denp[1], asrc, adst, easum,
                            wdict[wef_n[0]], wdict[wef_n[1]],
                            wdict[wef_n[2]], wdict[wef_n[3]],
                            wdict[wef_n[4]])
    part = _sc_gat_b(row, col, sarr, dfin, xs, zeros64)
    h = _tc_gat_final(part[0], part[1], sself, dfin, xs,
                      bias.reshape(1, HC))

  return _tc_pool(h, batch_p.reshape(NP, 1),
                  p['lin_w'], p['lin_b'].reshape(1, NCLS))
